# baseline (speedup 1.0000x reference)
import jax
import jax.numpy as jnp
from jax.experimental import pallas as pl

N_IN = 64
N_HID = 128
N_OUT = 16
BATCH = 16384


def _mlp_kernel(x_ref, w1_ref, w2_ref, b1_ref, b2_ref, r1_ref, r2_ref, o_ref):
    agg1 = jax.lax.dot_general(
        w1_ref[...], x_ref[...], (((1,), (0,)), ((), ())),
        preferred_element_type=jnp.float32,
    )
    h = jnp.tanh(b1_ref[...] + r1_ref[...] * agg1)
    agg2 = jax.lax.dot_general(
        w2_ref[...], h, (((1,), (0,)), ((), ())),
        preferred_element_type=jnp.float32,
    )
    o_ref[...] = jnp.tanh(b2_ref[...] + r2_ref[...] * agg2)


def kernel(inputs, W_ih, W_ho, b_hid, b_out, resp_hid, resp_out):
    TM = 8192
    grid = (BATCH // TM,)
    xT = inputs.T
    b1 = b_hid.reshape(N_HID, 1)
    r1 = resp_hid.reshape(N_HID, 1)
    b2 = b_out.reshape(N_OUT, 1)
    r2 = resp_out.reshape(N_OUT, 1)
    out_t = pl.pallas_call(
        _mlp_kernel,
        grid=grid,
        in_specs=[
            pl.BlockSpec((N_IN, TM), lambda i: (0, i)),
            pl.BlockSpec((N_HID, N_IN), lambda i: (0, 0)),
            pl.BlockSpec((N_OUT, N_HID), lambda i: (0, 0)),
            pl.BlockSpec((N_HID, 1), lambda i: (0, 0)),
            pl.BlockSpec((N_OUT, 1), lambda i: (0, 0)),
            pl.BlockSpec((N_HID, 1), lambda i: (0, 0)),
            pl.BlockSpec((N_OUT, 1), lambda i: (0, 0)),
        ],
        out_specs=pl.BlockSpec((N_OUT, TM), lambda i: (0, i)),
        out_shape=jax.ShapeDtypeStruct((N_OUT, BATCH), jnp.float32),
    )(xT, W_ih, W_ho, b1, b2, r1, r2)
    return out_t.T
